# Initial kernel scaffold; baseline (speedup 1.0000x reference)
import jax
import jax.numpy as jnp
from jax.experimental import pallas as pl


def kernel(title_idx, format_idx, studio_idx, source_idx, year_idx, title_table, format_table, studio_table, source_table, year_table):
    raise NotImplementedError("write your pallas kernel here")


# speedup vs baseline: 1.3081x; 1.3081x over previous
import functools

import jax
import jax.numpy as jnp
from jax import lax
from jax.experimental import pallas as pl
from jax.experimental.pallas import tpu as pltpu
from jax.experimental.pallas import tpu_sc as plsc

_B = 16384
_D = 64
_NUM_FEATURES = 5

_info = plsc.get_sparse_core_info()
_NC = _info.num_cores
_NS = _info.num_subcores
_NW = _NC * _NS
_BPW = _B // _NW


def _build():
    mesh = plsc.VectorSubcoreMesh(core_axis_name="c", subcore_axis_name="s")

    @functools.partial(
        pl.kernel,
        mesh=mesh,
        out_type=jax.ShapeDtypeStruct((_B, _NUM_FEATURES * _D), jnp.float32),
        scratch_types=[
            pltpu.VMEM((_BPW,), jnp.int32),
            pltpu.VMEM((_BPW, _D), jnp.float32),
            pltpu.SemaphoreType.DMA,
        ],
        compiler_params=pltpu.CompilerParams(use_tc_tiling_on_sc=False),
    )
    def sc_kernel(t_idx, f_idx, st_idx, so_idx, y_idx,
                  t_tab, f_tab, st_tab, so_tab, y_tab,
                  out, idx_v, rows_v, sem):
        wid = lax.axis_index("s") * _NC + lax.axis_index("c")
        base = wid * _BPW
        features = ((t_idx, t_tab), (f_idx, f_tab), (st_idx, st_tab),
                    (so_idx, so_tab), (y_idx, y_tab))
        for fi, (idx_hbm, tab_hbm) in enumerate(features):
            pltpu.sync_copy(idx_hbm.at[pl.ds(base, _BPW)], idx_v)
            pltpu.async_copy(tab_hbm.at[idx_v], rows_v, sem).wait()
            pltpu.sync_copy(rows_v, out.at[pl.ds(base, _BPW), pl.ds(fi * _D, _D)])

    return sc_kernel


_sc_kernel = _build()


@jax.jit
def kernel(title_idx, format_idx, studio_idx, source_idx, year_idx,
           title_table, format_table, studio_table, source_table, year_table):
    return _sc_kernel(title_idx, format_idx, studio_idx, source_idx, year_idx,
                      title_table, format_table, studio_table, source_table,
                      year_table)


# speedup vs baseline: 1.3268x; 1.0143x over previous
import functools

import jax
import jax.numpy as jnp
from jax import lax
from jax.experimental import pallas as pl
from jax.experimental.pallas import tpu as pltpu
from jax.experimental.pallas import tpu_sc as plsc

_B = 16384
_D = 64
_NUM_FEATURES = 5

_info = plsc.get_sparse_core_info()
_NC = _info.num_cores
_NS = _info.num_subcores
_NW = _NC * _NS
_BPW = _B // _NW


def _build():
    mesh = plsc.VectorSubcoreMesh(core_axis_name="c", subcore_axis_name="s")

    nbuf = 3

    @functools.partial(
        pl.kernel,
        mesh=mesh,
        out_type=jax.ShapeDtypeStruct((_B, _NUM_FEATURES * _D), jnp.float32),
        scratch_types=[
            pltpu.VMEM((_NUM_FEATURES, _BPW), jnp.int32),
            [pltpu.VMEM((_BPW, _D), jnp.float32) for _ in range(nbuf)],
            pltpu.SemaphoreType.DMA,
            [pltpu.SemaphoreType.DMA for _ in range(nbuf)],
            [pltpu.SemaphoreType.DMA for _ in range(nbuf)],
        ],
        compiler_params=pltpu.CompilerParams(use_tc_tiling_on_sc=False),
    )
    def sc_kernel(t_idx, f_idx, st_idx, so_idx, y_idx,
                  t_tab, f_tab, st_tab, so_tab, y_tab,
                  out, idx_v, bufs, isem, gsems, ssems):
        wid = lax.axis_index("s") * _NC + lax.axis_index("c")
        base = wid * _BPW
        idx_arrays = (t_idx, f_idx, st_idx, so_idx, y_idx)
        tables = (t_tab, f_tab, st_tab, so_tab, y_tab)

        icopies = [
            pltpu.async_copy(idx_arrays[fi].at[pl.ds(base, _BPW)],
                             idx_v.at[fi], isem)
            for fi in range(_NUM_FEATURES)
        ]
        for cp in icopies:
            cp.wait()

        gathers = [None] * _NUM_FEATURES
        scatters = [None] * _NUM_FEATURES

        def start_gather(fi):
            slot = fi % nbuf
            gathers[fi] = pltpu.async_copy(
                tables[fi].at[idx_v.at[fi]], bufs[slot], gsems[slot])

        def start_scatter(fi):
            slot = fi % nbuf
            gathers[fi].wait()
            scatters[fi] = pltpu.async_copy(
                bufs[slot],
                out.at[pl.ds(base, _BPW), pl.ds(fi * _D, _D)],
                ssems[slot])

        for fi in range(_NUM_FEATURES):
            if fi >= nbuf:
                scatters[fi - nbuf].wait()
            start_gather(fi)
            if fi >= 1:
                start_scatter(fi - 1)
        start_scatter(_NUM_FEATURES - 1)
        for fi in range(_NUM_FEATURES - nbuf, _NUM_FEATURES):
            scatters[fi].wait()

    return sc_kernel


_sc_kernel = _build()


@jax.jit
def kernel(title_idx, format_idx, studio_idx, source_idx, year_idx,
           title_table, format_table, studio_table, source_table, year_table):
    return _sc_kernel(title_idx, format_idx, studio_idx, source_idx, year_idx,
                      title_table, format_table, studio_table, source_table,
                      year_table)
